# Initial kernel scaffold; baseline (speedup 1.0000x reference)
import jax
import jax.numpy as jnp
from jax.experimental import pallas as pl


def kernel(node_feature, edge_index, edge_label_index, W1l, W1r, b1, W2l, W2r, b2):
    raise NotImplementedError("write your pallas kernel here")


# speedup vs baseline: 4.7161x; 4.7161x over previous
import functools

import jax
import jax.numpy as jnp
from jax import lax
from jax.experimental import pallas as pl
from jax.experimental.pallas import tpu as pltpu
from jax.experimental.pallas import tpu_sc as plsc

NC = 2
NS = 16
LN = 16

_f32 = jnp.float32


def _fill(ref, n, value):
    v = jnp.full((LN,), value, dtype=_f32)

    def body(i, _):
        ref[pl.ds(i * LN, LN)] = v
        return _

    lax.fori_loop(0, n // LN, body, None)


@functools.lru_cache(maxsize=None)
def _make_seg_kernel(n_nodes, n_edges, d, with_deg, n_pad):
    n_tiles = NC * NS
    per_tile = n_edges // n_tiles
    assert n_edges % n_tiles == 0
    CH = 80
    assert per_tile % CH == 0
    n_chunks = per_tile // CH
    ZR = 125
    rows_sub = 1000
    n_writers = n_nodes // rows_sub
    assert n_nodes % rows_sub == 0 and n_writers <= NS
    assert rows_sub % ZR == 0
    deg_sub = n_pad // NS
    assert n_pad % NS == 0 and deg_sub % LN == 0

    mesh = plsc.VectorSubcoreMesh(core_axis_name="c", subcore_axis_name="s")

    out_type = [jax.ShapeDtypeStruct((NC, n_nodes, d), _f32)]
    scratch = [
        pltpu.VMEM_SHARED((n_nodes, d), _f32),
        pltpu.VMEM((ZR, d), _f32),
        pltpu.VMEM((CH,), jnp.int32),
        pltpu.VMEM((CH,), jnp.int32),
        pltpu.VMEM((CH, d), _f32),
        pltpu.SemaphoreType.DMA,
    ]
    if with_deg:
        out_type.append(jax.ShapeDtypeStruct((NC, n_pad), _f32))
        scratch += [
            pltpu.VMEM_SHARED((n_pad,), _f32),
            pltpu.VMEM((deg_sub,), _f32),
            pltpu.VMEM((CH,), _f32),
        ]

    def body(y_hbm, src_hbm, dst_hbm, *refs):
        if with_deg:
            (acc_out, deg_out, acc_sh, zbuf, srcv, dstv, rows, sem,
             deg_sh, degz, ones) = refs
        else:
            acc_out, acc_sh, zbuf, srcv, dstv, rows, sem = refs
        c = lax.axis_index("c")
        s = lax.axis_index("s")

        zero16 = jnp.zeros((LN,), _f32)

        def zrow(i, _):
            for q in range(d // LN):
                zbuf[i, pl.ds(q * LN, LN)] = zero16
            return _

        lax.fori_loop(0, ZR, zrow, None)
        if with_deg:
            _fill(degz, deg_sub, 0.0)
            _fill(ones, CH, 1.0)

        nbase = s * rows_sub

        @pl.when(s < n_writers)
        def _zero():
            for k in range(rows_sub // ZR):
                pltpu.sync_copy(zbuf, acc_sh.at[pl.ds(nbase + k * ZR, ZR)])

        if with_deg:
            pltpu.sync_copy(degz, deg_sh.at[pl.ds(s * deg_sub, deg_sub)])
        plsc.subcore_barrier()

        ebase = (c * NS + s) * per_tile

        def chunk(j, _):
            off = ebase + j * CH
            pltpu.sync_copy(src_hbm.at[pl.ds(off, CH)], srcv)
            pltpu.sync_copy(dst_hbm.at[pl.ds(off, CH)], dstv)
            pltpu.async_copy(y_hbm.at[srcv], rows, sem).wait()
            pltpu.sync_copy(rows, acc_sh.at[dstv], add=True)
            if with_deg:
                pltpu.sync_copy(ones, deg_sh.at[dstv], add=True)
            return _

        lax.fori_loop(0, n_chunks, chunk, None)
        plsc.subcore_barrier()

        @pl.when(s < n_writers)
        def _writeout():
            pltpu.sync_copy(acc_sh.at[pl.ds(nbase, rows_sub)],
                            acc_out.at[c, pl.ds(nbase, rows_sub)])

        if with_deg:
            pltpu.sync_copy(deg_sh.at[pl.ds(s * deg_sub, deg_sub)],
                            deg_out.at[c, pl.ds(s * deg_sub, deg_sub)])

    return pl.kernel(body, out_type=out_type, mesh=mesh, scratch_types=scratch,
                     compiler_params=pltpu.CompilerParams(use_tc_tiling_on_sc=False),
                     name=f"sc_segsum_d{d}")


@functools.lru_cache(maxsize=None)
def _make_score_kernel(n_nodes, d, l_pad):
    n_tiles = NC * NS
    CH = 128
    per_tile = l_pad // n_tiles
    assert l_pad % (n_tiles * CH) == 0
    n_chunks = per_tile // CH

    mesh = plsc.VectorSubcoreMesh(core_axis_name="c", subcore_axis_name="s")
    scratch = [
        pltpu.VMEM((CH,), jnp.int32),
        pltpu.VMEM((CH,), jnp.int32),
        pltpu.VMEM((CH, d), _f32),
        pltpu.VMEM((CH, d), _f32),
        pltpu.VMEM((CH, LN), _f32),
        pltpu.SemaphoreType.DMA,
    ]

    def body(h_hbm, a_hbm, b_hbm, out_hbm, ia, ib, ra, rb, outv, sem):
        c = lax.axis_index("c")
        s = lax.axis_index("s")
        base = (c * NS + s) * per_tile

        def chunk(j, _):
            off = base + j * CH
            pltpu.sync_copy(a_hbm.at[pl.ds(off, CH)], ia)
            pltpu.sync_copy(b_hbm.at[pl.ds(off, CH)], ib)
            pltpu.async_copy(h_hbm.at[ia], ra, sem).wait()
            pltpu.async_copy(h_hbm.at[ib], rb, sem).wait()

            def edge(e, _):
                acc = ra[e, pl.ds(0, LN)] * rb[e, pl.ds(0, LN)]
                for q in range(1, d // LN):
                    acc = acc + (ra[e, pl.ds(q * LN, LN)]
                                 * rb[e, pl.ds(q * LN, LN)])
                outv[e, :] = acc
                return _

            lax.fori_loop(0, CH, edge, None)
            pltpu.sync_copy(outv, out_hbm.at[pl.ds(off, CH)])
            return _

        lax.fori_loop(0, n_chunks, chunk, None)

    return pl.kernel(body, out_type=jax.ShapeDtypeStruct((l_pad, LN), _f32),
                     mesh=mesh, scratch_types=scratch,
                     compiler_params=pltpu.CompilerParams(use_tc_tiling_on_sc=False),
                     name="sc_score")


def _tc_pre(x, wl, wr, b):
    n, din = x.shape
    dh = wl.shape[1]
    R = 2000
    assert n % R == 0

    def body(x_ref, wl_ref, wr_ref, b_ref, y_ref, r_ref):
        xb = x_ref[...]
        y_ref[...] = jnp.dot(xb, wl_ref[...], preferred_element_type=_f32)
        r_ref[...] = (jnp.dot(xb, wr_ref[...], preferred_element_type=_f32)
                      + b_ref[...])

    return pl.pallas_call(
        body,
        grid=(n // R,),
        in_specs=[
            pl.BlockSpec((R, din), lambda i: (i, 0)),
            pl.BlockSpec((din, dh), lambda i: (0, 0)),
            pl.BlockSpec((din, dh), lambda i: (0, 0)),
            pl.BlockSpec((1, dh), lambda i: (0, 0)),
        ],
        out_specs=[
            pl.BlockSpec((R, dh), lambda i: (i, 0)),
            pl.BlockSpec((R, dh), lambda i: (i, 0)),
        ],
        out_shape=[jax.ShapeDtypeStruct((n, dh), _f32)] * 2,
    )(x, wl, wr, b.reshape(1, -1))


def _tc_mid(p, degp, r1, wl, wr, b):
    _, n, dh = p.shape
    do = wl.shape[1]
    R = 2000
    assert n % R == 0

    def body(p_ref, dg_ref, r1_ref, wl_ref, wr_ref, b_ref, y_ref, r_ref):
        deg = dg_ref[:, 0:1] + dg_ref[:, 1:2]
        rdeg = 1.0 / jnp.maximum(deg, 1.0)
        agg = (p_ref[0] + p_ref[1]) * rdeg
        h1 = jnp.maximum(agg + r1_ref[...], 0.0)
        y_ref[...] = jnp.dot(h1, wl_ref[...], preferred_element_type=_f32)
        r_ref[...] = (jnp.dot(h1, wr_ref[...], preferred_element_type=_f32)
                      + b_ref[...])

    return pl.pallas_call(
        body,
        grid=(n // R,),
        in_specs=[
            pl.BlockSpec((NC, R, dh), lambda i: (0, i, 0)),
            pl.BlockSpec((R, NC), lambda i: (i, 0)),
            pl.BlockSpec((R, dh), lambda i: (i, 0)),
            pl.BlockSpec((dh, do), lambda i: (0, 0)),
            pl.BlockSpec((dh, do), lambda i: (0, 0)),
            pl.BlockSpec((1, do), lambda i: (0, 0)),
        ],
        out_specs=[
            pl.BlockSpec((R, do), lambda i: (i, 0)),
            pl.BlockSpec((R, do), lambda i: (i, 0)),
        ],
        out_shape=[jax.ShapeDtypeStruct((n, do), _f32)] * 2,
    )(p, degp, r1, wl, wr, b.reshape(1, -1))


def _tc_post(p, degp, r2):
    _, n, do = p.shape
    R = 2000
    assert n % R == 0

    def body(p_ref, dg_ref, r2_ref, h_ref):
        deg = dg_ref[:, 0:1] + dg_ref[:, 1:2]
        rdeg = 1.0 / jnp.maximum(deg, 1.0)
        h_ref[...] = (p_ref[0] + p_ref[1]) * rdeg + r2_ref[...]

    return pl.pallas_call(
        body,
        grid=(n // R,),
        in_specs=[
            pl.BlockSpec((NC, R, do), lambda i: (0, i, 0)),
            pl.BlockSpec((R, NC), lambda i: (i, 0)),
            pl.BlockSpec((R, do), lambda i: (i, 0)),
        ],
        out_specs=pl.BlockSpec((R, do), lambda i: (i, 0)),
        out_shape=jax.ShapeDtypeStruct((n, do), _f32),
    )(p, degp, r2)


def _tc_lane_reduce(parts):
    l_pad = parts.shape[0]
    R = 6400
    assert l_pad % R == 0

    def body(p_ref, o_ref):
        o_ref[...] = jnp.sum(p_ref[...], axis=1, keepdims=True)

    return pl.pallas_call(
        body,
        grid=(l_pad // R,),
        in_specs=[pl.BlockSpec((R, LN), lambda i: (i, 0))],
        out_specs=pl.BlockSpec((R, 1), lambda i: (i, 0)),
        out_shape=jax.ShapeDtypeStruct((l_pad, 1), _f32),
    )(parts)


def kernel(node_feature, edge_index, edge_label_index, W1l, W1r, b1, W2l, W2r, b2):
    n, din = node_feature.shape
    dh = W1l.shape[1]
    do = W2l.shape[1]
    n_edges = edge_index.shape[1]
    n_lbl = edge_label_index.shape[1]

    src = edge_index[0]
    dst = edge_index[1]

    n_pad = ((n + NS * LN - 1) // (NS * LN)) * (NS * LN)

    y1, r1 = _tc_pre(node_feature, W1l, W1r, b1)
    p1, degp = _make_seg_kernel(n, n_edges, dh, True, n_pad)(y1, src, dst)
    degp = degp.T[:n]
    y2, r2 = _tc_mid(p1, degp, r1, W2l, W2r, b2)

    (p2,) = _make_seg_kernel(n, n_edges, do, False, n_pad)(y2, src, dst)
    h2 = _tc_post(p2, degp, r2)

    tile_ch = NC * NS * 128
    l_pad = ((n_lbl + tile_ch - 1) // tile_ch) * tile_ch
    a = jnp.pad(edge_label_index[0], (0, l_pad - n_lbl))
    b = jnp.pad(edge_label_index[1], (0, l_pad - n_lbl))
    parts = _make_score_kernel(n, do, l_pad)(h2, a, b)
    pred = _tc_lane_reduce(parts)
    return pred.reshape(-1)[:n_lbl]
